# Initial kernel scaffold; baseline (speedup 1.0000x reference)
import jax
import jax.numpy as jnp
from jax.experimental import pallas as pl


def kernel(x, edge_index, batch, gcn1_w, gcn1_b, gcn2_w, gcn2_b, gcn3_w, gcn3_b, gcn4_w, gcn4_b, ecn1_w1, ecn1_b1, ecn1_w2, ecn1_b2, ecn2_w1, ecn2_b1, ecn2_w2, ecn2_b2, fc1_w, fc1_b, out_w, out_b):
    raise NotImplementedError("write your pallas kernel here")


# speedup vs baseline: 1.7588x; 1.7588x over previous
import jax
import jax.numpy as jnp
from jax.experimental import pallas as pl

N = 10000
G = 64

jnp_orig_matmul = jnp.matmul


def _mm(a, b):
    return jax.lax.dot(a, b, precision=jax.lax.Precision.HIGHEST)


def _gcn(x, row, col, norm, W, b):
    xw = _mm(x, W)
    msg = xw[row] * norm[:, None]
    return jax.ops.segment_sum(msg, col, num_segments=N) + b


def _edge_conv(x, row, col, W1, b1, W2, b2):
    F = x.shape[1]
    W1a, W1b = W1[:F], W1[F:]
    u = _mm(x, W1a - W1b)
    v = _mm(x, W1b)
    h = _mm(jax.nn.relu(u[col] + v[row] + b1), W2) + b2
    out = jax.ops.segment_max(h, col, num_segments=N)
    return jnp.where(jnp.isfinite(out), out, 0.0)


def kernel(x, edge_index, batch, gcn1_w, gcn1_b, gcn2_w, gcn2_b, gcn3_w, gcn3_b, gcn4_w, gcn4_b, ecn1_w1, ecn1_b1, ecn1_w2, ecn1_b2, ecn2_w1, ecn2_b1, ecn2_w2, ecn2_b2, fc1_w, fc1_b, out_w, out_b):
    row, col = edge_index[0], edge_index[1]
    deg = jnp.ones((N,), jnp.float32).at[col].add(1.0)
    dinv = jax.lax.rsqrt(deg)
    relu = jax.nn.relu

    def gcn(x, W, b):
        y = _mm(x, W) * dinv[:, None]
        s = y + jax.ops.segment_sum(y[row], col, num_segments=N)
        return relu(s * dinv[:, None] + b)

    xg = gcn(x, gcn1_w, gcn1_b)
    xg = gcn(xg, gcn2_w, gcn2_b)
    xg = gcn(xg, gcn3_w, gcn3_b)
    xg = gcn(xg, gcn4_w, gcn4_b)
    xe = relu(_edge_conv(x, row, col, ecn1_w1, ecn1_b1, ecn1_w2, ecn1_b2))
    xe = relu(_edge_conv(xe, row, col, ecn2_w1, ecn2_b1, ecn2_w2, ecn2_b2))
    x_cat = jnp.concatenate([xg, xe], axis=1)
    g = jax.ops.segment_sum(x_cat, batch, num_segments=G)
    g = relu(_mm(g, fc1_w) + fc1_b)
    return _mm(g, out_w) + out_b


# speedup vs baseline: 1.8835x; 1.0709x over previous
import functools

import jax
import jax.numpy as jnp
from jax import lax
from jax.experimental import pallas as pl
from jax.experimental.pallas import tpu as pltpu
from jax.experimental.pallas import tpu_sc as plsc

N = 10000
E = 320000
G = 64

NW = 32
RNG = 320
NP = NW * RNG
CH = 4000
CHB = CH + 16
K = 64
NCHK = E // CH
EBUFP = E + NW * (7 * NCHK + 64)
EBUF = EBUFP + NW * CHB

_mesh = plsc.VectorSubcoreMesh(core_axis_name="c", subcore_axis_name="s")
_sc_params = pltpu.CompilerParams(needs_layout_passes=False)


def _wid():
    return lax.axis_index("s") * 2 + lax.axis_index("c")


def _iota16():
    return lax.iota(jnp.int32, 16)


def _splat(x):
    return jnp.full((16,), x, jnp.int32)


def _scal(v):
    return jnp.max(v)


def _prefix(cntv, w):
    base_v = jnp.zeros((16,), jnp.int32)
    for t in range(NW):
        ct = cntv[pl.ds(t * 16, 16)]
        base_v = base_v + jnp.where(t < w, ct, 0)
    return _scal(base_v)


def _my_count(cntv, w):
    return _scal(plsc.load_gather(cntv, [_splat(w * 16) + _iota16()]))


@functools.partial(
    pl.kernel,
    out_type=jax.ShapeDtypeStruct((NW * 16,), jnp.int32),
    mesh=_mesh,
    compiler_params=_sc_params,
    scratch_types=[pltpu.VMEM((CH,), jnp.int32),
                   pltpu.VMEM((16,), jnp.int32)],
)
def _filter_count(col_hbm, cnt_hbm, colv, outv):
    w = _wid()
    lo = w * RNG
    hi = lo + RNG

    def chunk_body(ci, padded):
        pltpu.sync_copy(col_hbm.at[pl.ds(ci * CH, CH)], colv)

        def grp_body(gi, c):
            cv = plsc.load_gather(colv, [_splat(gi * 16) + _iota16()])
            m = (cv >= lo) & (cv < hi)
            return c + _scal(plsc.all_reduce_population_count(m))

        c = lax.fori_loop(0, CH // 16, grp_body, jnp.int32(0))
        return padded + ((c + 7) // 8) * 8

    padded = lax.fori_loop(0, NCHK, chunk_body, jnp.int32(0))
    padded = ((padded + 63) // 64) * 64
    outv[...] = jnp.broadcast_to(padded, (16,))
    pltpu.sync_copy(outv, cnt_hbm.at[pl.ds(w * 16, 16)])


@functools.partial(
    pl.kernel,
    out_type=(
        jax.ShapeDtypeStruct((EBUF,), jnp.int32),
        jax.ShapeDtypeStruct((EBUF,), jnp.int32),
    ),
    mesh=_mesh,
    compiler_params=_sc_params,
    scratch_types=[
        pltpu.VMEM((CH,), jnp.int32),
        pltpu.VMEM((CH,), jnp.int32),
        pltpu.VMEM((NW * 16,), jnp.int32),
        pltpu.VMEM((CHB,), jnp.int32),
        pltpu.VMEM((CHB,), jnp.int32),
    ],
)
def _filter_compact(row_hbm, col_hbm, cnt_hbm, rowo_hbm, colo_hbm,
                    rowv, colv, cntv, rbuf, cbuf):
    w = _wid()
    lo = w * RNG
    hi = lo + RNG
    pltpu.sync_copy(cnt_hbm, cntv)
    glob0 = _prefix(cntv, w) + w * CHB

    def chunk_body(ci, glob):
        pltpu.sync_copy(col_hbm.at[pl.ds(ci * CH, CH)], colv)
        pltpu.sync_copy(row_hbm.at[pl.ds(ci * CH, CH)], rowv)

        def grp_body(gi, loc):
            idx = _splat(gi * 16) + _iota16()
            cv = plsc.load_gather(colv, [idx])
            rv = plsc.load_gather(rowv, [idx])
            m = (cv >= lo) & (cv < hi)
            plsc.store_compressed(rbuf.at[pl.ds(loc, 16)], rv, mask=m)
            plsc.store_compressed(cbuf.at[pl.ds(loc, 16)], cv - lo, mask=m)
            return loc + _scal(plsc.all_reduce_population_count(m))

        loc = lax.fori_loop(0, CH // 16, grp_body, jnp.int32(0))
        p = (8 - lax.rem(loc, 8)) & 7
        pm = _iota16() < p
        plsc.store_scatter(rbuf, [_splat(loc) + _iota16()],
                           jnp.zeros((16,), jnp.int32), mask=pm)
        plsc.store_scatter(cbuf, [_splat(loc) + _iota16()],
                           _splat(RNG), mask=pm)
        globa = pl.multiple_of(glob, 8)
        pltpu.sync_copy(rbuf, rowo_hbm.at[pl.ds(globa, CHB)])
        pltpu.sync_copy(cbuf, colo_hbm.at[pl.ds(globa, CHB)])
        return glob + loc + p

    glob = lax.fori_loop(0, NCHK, chunk_body, glob0)
    for q in range(4):
        rbuf[pl.ds(q * 16, 16)] = jnp.zeros((16,), jnp.int32)
        cbuf[pl.ds(q * 16, 16)] = _splat(RNG)
    globa = pl.multiple_of(glob, 8)
    pltpu.sync_copy(rbuf.at[pl.ds(0, 64)], rowo_hbm.at[pl.ds(globa, 64)])
    pltpu.sync_copy(cbuf.at[pl.ds(0, 64)], colo_hbm.at[pl.ds(globa, 64)])


@functools.partial(
    pl.kernel,
    out_type=jax.ShapeDtypeStruct((NP, 256), jnp.float32),
    mesh=_mesh,
    compiler_params=_sc_params,
    scratch_types=[
        pltpu.VMEM((RNG + 1, 256), jnp.float32),
        pltpu.VMEM((K, 256), jnp.float32),
        pltpu.VMEM((K,), jnp.int32),
        pltpu.VMEM((K,), jnp.int32),
        pltpu.VMEM((NW * 16,), jnp.int32),
        pltpu.SemaphoreType.DMA,
    ],
)
def _gcn_agg(y_hbm, rowo_hbm, colo_hbm, cnt_hbm, out_hbm,
             acc, rows, ridx, cidx, cntv, sem):
    w = _wid()
    pltpu.sync_copy(cnt_hbm, cntv)
    base_w = _prefix(cntv, w) + w * CHB
    padded_w = _my_count(cntv, w)

    pltpu.sync_copy(y_hbm.at[pl.ds(w * RNG, RNG)], acc.at[pl.ds(0, RNG)])

    def chunk_body(g, _):
        be = pl.multiple_of(base_w + g * K, 8)
        pltpu.sync_copy(rowo_hbm.at[pl.ds(be, K)], ridx)
        pltpu.sync_copy(colo_hbm.at[pl.ds(be, K)], cidx)
        for q in range(K // 16):
            ridx[pl.ds(q * 16, 16)] = jnp.clip(ridx[pl.ds(q * 16, 16)], 0, NP - 1)
        pltpu.async_copy(y_hbm.at[ridx], rows, sem).wait()

        def edge_body(j, _):
            csp = plsc.load_gather(cidx, [_splat(j)])
            for f in range(16):
                v = plsc.load_gather(rows, [_splat(j), _splat(f * 16) + _iota16()])
                plsc.addupdate_scatter(acc, [csp, _splat(f * 16) + _iota16()], v)
            return 0

        lax.fori_loop(0, K, edge_body, 0)
        return 0

    lax.fori_loop(0, padded_w // K, chunk_body, 0)
    pltpu.sync_copy(acc.at[pl.ds(0, RNG)], out_hbm.at[pl.ds(w * RNG, RNG)])


@functools.partial(
    pl.kernel,
    out_type=jax.ShapeDtypeStruct((NP,), jnp.float32),
    mesh=_mesh,
    compiler_params=_sc_params,
    scratch_types=[
        pltpu.VMEM((RNG + 16,), jnp.float32),
        pltpu.VMEM((K,), jnp.int32),
        pltpu.VMEM((NW * 16,), jnp.int32),
    ],
)
def _deg_kernel(colo_hbm, cnt_hbm, deg_hbm, acc, cidx, cntv):
    w = _wid()
    pltpu.sync_copy(cnt_hbm, cntv)
    base_w = _prefix(cntv, w) + w * CHB
    padded_w = _my_count(cntv, w)

    ones = jnp.ones((16,), jnp.float32)
    for q in range((RNG + 16) // 16):
        acc[pl.ds(q * 16, 16)] = ones
    m0 = _iota16() < 1

    def chunk_body(g, _):
        be = pl.multiple_of(base_w + g * K, 8)
        pltpu.sync_copy(colo_hbm.at[pl.ds(be, K)], cidx)

        def edge_body(j, _):
            csp = plsc.load_gather(cidx, [_splat(j)])
            plsc.addupdate_scatter(acc, [csp], ones, mask=m0)
            return 0

        lax.fori_loop(0, K, edge_body, 0)
        return 0

    lax.fori_loop(0, padded_w // K, chunk_body, 0)
    pltpu.sync_copy(acc.at[pl.ds(0, RNG)], deg_hbm.at[pl.ds(w * RNG, RNG)])


@functools.partial(
    pl.kernel,
    out_type=jax.ShapeDtypeStruct((EBUFP, 256), jnp.float32),
    mesh=_mesh,
    compiler_params=_sc_params,
    scratch_types=[
        pltpu.VMEM((K, 256), jnp.float32),
        pltpu.VMEM((K, 256), jnp.float32),
        pltpu.VMEM((K,), jnp.int32),
        pltpu.VMEM((K,), jnp.int32),
        pltpu.VMEM((NW * 16,), jnp.int32),
        pltpu.SemaphoreType.DMA,
        pltpu.SemaphoreType.DMA,
    ],
)
def _ec_pre(u_hbm, v_hbm, rowo_hbm, colo_hbm, cnt_hbm, pre_hbm,
            ru, rv, ridx, cidx, cntv, sem1, sem2):
    w = _wid()
    pltpu.sync_copy(cnt_hbm, cntv)
    basep = _prefix(cntv, w)
    base_w = basep + w * CHB
    padded_w = _my_count(cntv, w)

    def chunk_body(g, _):
        be = pl.multiple_of(base_w + g * K, 8)
        pltpu.sync_copy(rowo_hbm.at[pl.ds(be, K)], ridx)
        pltpu.sync_copy(colo_hbm.at[pl.ds(be, K)], cidx)
        for q in range(K // 16):
            ridx[pl.ds(q * 16, 16)] = jnp.clip(ridx[pl.ds(q * 16, 16)], 0, NP - 1)
            cidx[pl.ds(q * 16, 16)] = jnp.clip(
                cidx[pl.ds(q * 16, 16)] + w * RNG, 0, NP - 1)
        cu = pltpu.async_copy(u_hbm.at[cidx], ru, sem1)
        cv = pltpu.async_copy(v_hbm.at[ridx], rv, sem2)
        cu.wait()
        cv.wait()

        def edge_body(j, _):
            for f in range(16):
                ix = [_splat(j), _splat(f * 16) + _iota16()]
                plsc.addupdate_scatter(ru, ix, plsc.load_gather(rv, ix))
            return 0

        lax.fori_loop(0, K, edge_body, 0)
        ob = pl.multiple_of(basep + g * K, 8)
        pltpu.sync_copy(ru, pre_hbm.at[pl.ds(ob, K)])
        return 0

    lax.fori_loop(0, padded_w // K, chunk_body, 0)


@functools.partial(
    pl.kernel,
    out_type=jax.ShapeDtypeStruct((NP, 256), jnp.float32),
    mesh=_mesh,
    compiler_params=_sc_params,
    scratch_types=[
        pltpu.VMEM((RNG + 1, 256), jnp.float32),
        pltpu.VMEM((K, 256), jnp.float32),
        pltpu.VMEM((K,), jnp.int32),
        pltpu.VMEM((NW * 16,), jnp.int32),
    ],
)
def _ec_max(h_hbm, colo_hbm, cnt_hbm, xe_hbm, acc, rows, cidx, cntv):
    w = _wid()
    pltpu.sync_copy(cnt_hbm, cntv)
    basep = _prefix(cntv, w)
    base_w = basep + w * CHB
    padded_w = _my_count(cntv, w)

    z = jnp.zeros((16,), jnp.float32)
    for r in range(RNG + 1):
        for f in range(16):
            acc[r, pl.ds(f * 16, 16)] = z

    def chunk_body(g, _):
        be = pl.multiple_of(base_w + g * K, 8)
        pltpu.sync_copy(colo_hbm.at[pl.ds(be, K)], cidx)
        hb = pl.multiple_of(basep + g * K, 8)
        pltpu.sync_copy(h_hbm.at[pl.ds(hb, K)], rows)

        def edge_body(j, _):
            csp = plsc.load_gather(cidx, [_splat(j)])
            for f in range(16):
                fx = _splat(f * 16) + _iota16()
                a = plsc.load_gather(acc, [csp, fx])
                v = plsc.load_gather(rows, [_splat(j), fx])
                plsc.store_scatter(acc, [csp, fx], jnp.maximum(a, v))
            return 0

        lax.fori_loop(0, K, edge_body, 0)
        return 0

    lax.fori_loop(0, padded_w // K, chunk_body, 0)
    pltpu.sync_copy(acc.at[pl.ds(0, RNG)], xe_hbm.at[pl.ds(w * RNG, RNG)])


def _dotf(a, b):
    return lax.dot(a, b, precision=lax.Precision.HIGHEST,
                   preferred_element_type=jnp.float32)


def _mm_body(pre_mode, post_scale, out_bias, x_ref, w_ref, bp_ref, bo_ref,
             dv_ref, o_ref):
    x = x_ref[...]
    if pre_mode == 1:
        x = jnp.maximum(x * dv_ref[..., 0:1] + bp_ref[...], 0.0)
    elif pre_mode == 2:
        x = jnp.maximum(x, 0.0)
    y = _dotf(x, w_ref[...])
    if post_scale:
        y = y * dv_ref[..., 0:1]
    if out_bias:
        y = y + bo_ref[...]
    o_ref[...] = y


def _mm(x, w, b_prev=None, b_out=None, dinv=None, pre_mode=0,
        post_scale=False):
    m, kd = x.shape
    f = w.shape[1]
    assert m % 512 == 0
    bp = jnp.zeros((1, f), jnp.float32) if b_prev is None else b_prev.reshape(1, f)
    bo = jnp.zeros((1, f), jnp.float32) if b_out is None else b_out.reshape(1, f)
    dv = (jnp.zeros((512, 128), jnp.float32) if dinv is None else dinv)
    dspec = (pl.BlockSpec((512, 128), lambda i: (0, 0)) if dinv is None
             else pl.BlockSpec((512, 128), lambda i: (i, 0)))
    body = functools.partial(_mm_body, pre_mode, post_scale, b_out is not None)
    return pl.pallas_call(
        body,
        grid=(m // 512,),
        in_specs=[
            pl.BlockSpec((512, kd), lambda i: (i, 0)),
            pl.BlockSpec((kd, f), lambda i: (0, 0)),
            pl.BlockSpec((1, f), lambda i: (0, 0)),
            pl.BlockSpec((1, f), lambda i: (0, 0)),
            dspec,
        ],
        out_specs=pl.BlockSpec((512, f), lambda i: (i, 0)),
        out_shape=jax.ShapeDtypeStruct((m, f), jnp.float32),
    )(x, w, bp, bo, dv)


def _rsqrt_body(d_ref, o_ref):
    o_ref[...] = lax.rsqrt(d_ref[...])


def _rsqrt(deg128):
    return pl.pallas_call(
        _rsqrt_body,
        grid=(NP // 512,),
        in_specs=[pl.BlockSpec((512, 128), lambda i: (i, 0))],
        out_specs=pl.BlockSpec((512, 128), lambda i: (i, 0)),
        out_shape=jax.ShapeDtypeStruct((NP, 128), jnp.float32),
    )(deg128)


def _pool_body(s4_ref, xe_ref, dv_ref, b4_ref, bt_ref, fw_ref, fb_ref,
               ow_ref, ob_ref, o_ref, acc):
    i = pl.program_id(0)

    @pl.when(i == 0)
    def _():
        acc[...] = jnp.zeros_like(acc)

    xg = jnp.maximum(s4_ref[...] * dv_ref[..., 0:1] + b4_ref[...], 0.0)
    xe = xe_ref[...]
    bt = bt_ref[0, 0, :]
    onehot = (bt[None, :] == lax.broadcasted_iota(jnp.int32, (G, 512), 0)
              ).astype(jnp.float32)
    acc[:, 0:256] += _dotf(onehot, xg)
    acc[:, 256:512] += _dotf(onehot, xe)

    @pl.when(i == NP // 512 - 1)
    def _():
        g = jnp.maximum(_dotf(acc[...], fw_ref[...]) + fb_ref[...], 0.0)
        o_ref[...] = _dotf(g, ow_ref[...]) + ob_ref[0, 0]


def _pool(s4, xe, dinv, b4, batch3, fc1_w, fc1_b, out_w, out_b):
    return pl.pallas_call(
        _pool_body,
        grid=(NP // 512,),
        in_specs=[
            pl.BlockSpec((512, 256), lambda i: (i, 0)),
            pl.BlockSpec((512, 256), lambda i: (i, 0)),
            pl.BlockSpec((512, 128), lambda i: (i, 0)),
            pl.BlockSpec((1, 256), lambda i: (0, 0)),
            pl.BlockSpec((1, 1, 512), lambda i: (i, 0, 0)),
            pl.BlockSpec((512, 256), lambda i: (0, 0)),
            pl.BlockSpec((1, 256), lambda i: (0, 0)),
            pl.BlockSpec((256, 128), lambda i: (0, 0)),
            pl.BlockSpec((1, 1), lambda i: (0, 0)),
        ],
        out_specs=pl.BlockSpec((G, 128), lambda i: (0, 0)),
        out_shape=jax.ShapeDtypeStruct((G, 128), jnp.float32),
        scratch_shapes=[pltpu.VMEM((G, 512), jnp.float32)],
    )(s4, xe, dinv, b4.reshape(1, 256), batch3, fc1_w, fc1_b.reshape(1, 256),
      jnp.broadcast_to(out_w, (256, 128)), out_b.reshape(1, 1))


def kernel(x, edge_index, batch, gcn1_w, gcn1_b, gcn2_w, gcn2_b, gcn3_w, gcn3_b, gcn4_w, gcn4_b, ecn1_w1, ecn1_b1, ecn1_w2, ecn1_b2, ecn2_w1, ecn2_b1, ecn2_w2, ecn2_b2, fc1_w, fc1_b, out_w, out_b):
    row, col = edge_index[0], edge_index[1]

    cnts = _filter_count(col)
    rowo, colo = _filter_compact(row, col, cnts)
    deg = _deg_kernel(colo, cnts)
    dinv = _rsqrt(jnp.broadcast_to(deg[:, None], (NP, 128)))

    xp = jnp.zeros((NP, 128), jnp.float32).at[:N].set(x)

    y = _mm(xp, gcn1_w, dinv=dinv, post_scale=True)
    s = _gcn_agg(y, rowo, colo, cnts)
    for (w_l, b_prev) in ((gcn2_w, gcn1_b), (gcn3_w, gcn2_b), (gcn4_w, gcn3_b)):
        y = _mm(s, w_l, b_prev=b_prev, dinv=dinv, pre_mode=1, post_scale=True)
        s = _gcn_agg(y, rowo, colo, cnts)

    def edge_conv(src, w1, b1, w2, b2):
        fin = src.shape[1]
        u = _mm(src, w1[:fin] - w1[fin:], b_out=b1)
        v = _mm(src, w1[fin:])
        pre = _ec_pre(u, v, rowo, colo, cnts)
        h = _mm(pre, w2, b_out=b2, pre_mode=2)
        return _ec_max(h, colo, cnts)

    xe = edge_conv(xp, ecn1_w1, ecn1_b1, ecn1_w2, ecn1_b2)
    xe = edge_conv(xe, ecn2_w1, ecn2_b1, ecn2_w2, ecn2_b2)

    batch_p = jnp.full((NP,), G + 7, jnp.int32).at[:N].set(batch.astype(jnp.int32))
    out = _pool(s, xe, dinv, gcn4_b, batch_p.reshape(NP // 512, 1, 512),
                fc1_w, fc1_b, out_w, out_b)
    return out[:, :1]


# speedup vs baseline: 2.3835x; 1.2655x over previous
import functools

import jax
import jax.numpy as jnp
from jax import lax
from jax.experimental import pallas as pl
from jax.experimental.pallas import tpu as pltpu
from jax.experimental.pallas import tpu_sc as plsc

N = 10000
E = 320000
G = 64

NW = 32
RNG = 320
NP = NW * RNG
CH = 4000
CHB = CH + 16
K = 64
NCHK = E // CH
EBUFP = E + NW * (7 * NCHK + 64)
EBUF = EBUFP + NW * CHB

_mesh = plsc.VectorSubcoreMesh(core_axis_name="c", subcore_axis_name="s")
_sc_params = pltpu.CompilerParams(needs_layout_passes=False)


def _wid():
    return lax.axis_index("s") * 2 + lax.axis_index("c")


def _iota16():
    return lax.iota(jnp.int32, 16)


def _splat(x):
    return jnp.full((16,), x, jnp.int32)


def _scal(v):
    return jnp.max(v)


def _prefix(cntv, w):
    base_v = jnp.zeros((16,), jnp.int32)
    for t in range(NW):
        ct = cntv[pl.ds(t * 16, 16)]
        base_v = base_v + jnp.where(t < w, ct, 0)
    return _scal(base_v)


def _my_count(cntv, w):
    return _scal(plsc.load_gather(cntv, [_splat(w * 16) + _iota16()]))


@functools.partial(
    pl.kernel,
    out_type=jax.ShapeDtypeStruct((NW * 16,), jnp.int32),
    mesh=_mesh,
    compiler_params=_sc_params,
    scratch_types=[pltpu.VMEM((CH,), jnp.int32),
                   pltpu.VMEM((16,), jnp.int32)],
)
def _filter_count(col_hbm, cnt_hbm, colv, outv):
    w = _wid()
    lo = w * RNG
    hi = lo + RNG

    def chunk_body(ci, padded):
        pltpu.sync_copy(col_hbm.at[pl.ds(ci * CH, CH)], colv)

        def grp_body(gi, c):
            cv = plsc.load_gather(colv, [_splat(gi * 16) + _iota16()])
            m = (cv >= lo) & (cv < hi)
            return c + _scal(plsc.all_reduce_population_count(m))

        c = lax.fori_loop(0, CH // 16, grp_body, jnp.int32(0))
        return padded + ((c + 7) // 8) * 8

    padded = lax.fori_loop(0, NCHK, chunk_body, jnp.int32(0))
    padded = ((padded + 63) // 64) * 64
    outv[...] = jnp.broadcast_to(padded, (16,))
    pltpu.sync_copy(outv, cnt_hbm.at[pl.ds(w * 16, 16)])


@functools.partial(
    pl.kernel,
    out_type=(
        jax.ShapeDtypeStruct((EBUF,), jnp.int32),
        jax.ShapeDtypeStruct((EBUF,), jnp.int32),
    ),
    mesh=_mesh,
    compiler_params=_sc_params,
    scratch_types=[
        pltpu.VMEM((CH,), jnp.int32),
        pltpu.VMEM((CH,), jnp.int32),
        pltpu.VMEM((NW * 16,), jnp.int32),
        pltpu.VMEM((CHB,), jnp.int32),
        pltpu.VMEM((CHB,), jnp.int32),
    ],
)
def _filter_compact(row_hbm, col_hbm, cnt_hbm, rowo_hbm, colo_hbm,
                    rowv, colv, cntv, rbuf, cbuf):
    w = _wid()
    lo = w * RNG
    hi = lo + RNG
    pltpu.sync_copy(cnt_hbm, cntv)
    glob0 = _prefix(cntv, w) + w * CHB

    def chunk_body(ci, glob):
        pltpu.sync_copy(col_hbm.at[pl.ds(ci * CH, CH)], colv)
        pltpu.sync_copy(row_hbm.at[pl.ds(ci * CH, CH)], rowv)

        def grp_body(gi, loc):
            idx = _splat(gi * 16) + _iota16()
            cv = plsc.load_gather(colv, [idx])
            rv = plsc.load_gather(rowv, [idx])
            m = (cv >= lo) & (cv < hi)
            plsc.store_compressed(rbuf.at[pl.ds(loc, 16)], rv, mask=m)
            plsc.store_compressed(cbuf.at[pl.ds(loc, 16)], cv - lo, mask=m)
            return loc + _scal(plsc.all_reduce_population_count(m))

        loc = lax.fori_loop(0, CH // 16, grp_body, jnp.int32(0))
        p = (8 - lax.rem(loc, 8)) & 7
        pm = _iota16() < p
        plsc.store_scatter(rbuf, [_splat(loc) + _iota16()],
                           jnp.zeros((16,), jnp.int32), mask=pm)
        plsc.store_scatter(cbuf, [_splat(loc) + _iota16()],
                           _splat(RNG), mask=pm)
        globa = pl.multiple_of(glob, 8)
        pltpu.sync_copy(rbuf, rowo_hbm.at[pl.ds(globa, CHB)])
        pltpu.sync_copy(cbuf, colo_hbm.at[pl.ds(globa, CHB)])
        return glob + loc + p

    glob = lax.fori_loop(0, NCHK, chunk_body, glob0)
    for q in range(4):
        rbuf[pl.ds(q * 16, 16)] = jnp.zeros((16,), jnp.int32)
        cbuf[pl.ds(q * 16, 16)] = _splat(RNG)
    globa = pl.multiple_of(glob, 8)
    pltpu.sync_copy(rbuf.at[pl.ds(0, 64)], rowo_hbm.at[pl.ds(globa, 64)])
    pltpu.sync_copy(cbuf.at[pl.ds(0, 64)], colo_hbm.at[pl.ds(globa, 64)])


@functools.partial(
    pl.kernel,
    out_type=jax.ShapeDtypeStruct((NP, 256), jnp.float32),
    mesh=_mesh,
    compiler_params=_sc_params,
    scratch_types=[
        pltpu.VMEM((RNG + 1, 256), jnp.float32),
        pltpu.VMEM((K, 256), jnp.float32),
        pltpu.VMEM((K,), jnp.int32),
        pltpu.VMEM((K,), jnp.int32),
        pltpu.VMEM((NW * 16,), jnp.int32),
        pltpu.SemaphoreType.DMA,
    ],
)
def _gcn_agg(y_hbm, rowo_hbm, colo_hbm, cnt_hbm, out_hbm,
             acc, rows, ridx, cidx, cntv, sem):
    w = _wid()
    pltpu.sync_copy(cnt_hbm, cntv)
    base_w = _prefix(cntv, w) + w * CHB
    padded_w = _my_count(cntv, w)

    pltpu.sync_copy(y_hbm.at[pl.ds(w * RNG, RNG)], acc.at[pl.ds(0, RNG)])

    def chunk_body(g, _):
        be = pl.multiple_of(base_w + g * K, 8)
        pltpu.sync_copy(rowo_hbm.at[pl.ds(be, K)], ridx)
        pltpu.sync_copy(colo_hbm.at[pl.ds(be, K)], cidx)
        for q in range(K // 16):
            ridx[pl.ds(q * 16, 16)] = jnp.clip(ridx[pl.ds(q * 16, 16)], 0, NP - 1)
        pltpu.async_copy(y_hbm.at[ridx], rows, sem).wait()

        @plsc.parallel_loop(0, K, 1, unroll=2)
        def edge_body(j):
            csp = plsc.load_gather(cidx, [_splat(j)])
            for f in range(16):
                v = plsc.load_gather(rows, [_splat(j), _splat(f * 16) + _iota16()])
                plsc.addupdate_scatter(acc, [csp, _splat(f * 16) + _iota16()], v)

        return 0

    lax.fori_loop(0, padded_w // K, chunk_body, 0)
    pltpu.sync_copy(acc.at[pl.ds(0, RNG)], out_hbm.at[pl.ds(w * RNG, RNG)])


@functools.partial(
    pl.kernel,
    out_type=jax.ShapeDtypeStruct((NP,), jnp.float32),
    mesh=_mesh,
    compiler_params=_sc_params,
    scratch_types=[
        pltpu.VMEM((RNG + 16,), jnp.float32),
        pltpu.VMEM((K,), jnp.int32),
        pltpu.VMEM((NW * 16,), jnp.int32),
    ],
)
def _deg_kernel(colo_hbm, cnt_hbm, deg_hbm, acc, cidx, cntv):
    w = _wid()
    pltpu.sync_copy(cnt_hbm, cntv)
    base_w = _prefix(cntv, w) + w * CHB
    padded_w = _my_count(cntv, w)

    ones = jnp.ones((16,), jnp.float32)
    for q in range((RNG + 16) // 16):
        acc[pl.ds(q * 16, 16)] = ones
    m0 = _iota16() < 1

    def chunk_body(g, _):
        be = pl.multiple_of(base_w + g * K, 8)
        pltpu.sync_copy(colo_hbm.at[pl.ds(be, K)], cidx)

        @plsc.parallel_loop(0, K, 1, unroll=4)
        def edge_body(j):
            csp = plsc.load_gather(cidx, [_splat(j)])
            plsc.addupdate_scatter(acc, [csp], ones, mask=m0)

        return 0

    lax.fori_loop(0, padded_w // K, chunk_body, 0)
    pltpu.sync_copy(acc.at[pl.ds(0, RNG)], deg_hbm.at[pl.ds(w * RNG, RNG)])


@functools.partial(
    pl.kernel,
    out_type=jax.ShapeDtypeStruct((EBUFP, 256), jnp.float32),
    mesh=_mesh,
    compiler_params=_sc_params,
    scratch_types=[
        pltpu.VMEM((K, 256), jnp.float32),
        pltpu.VMEM((K, 256), jnp.float32),
        pltpu.VMEM((K,), jnp.int32),
        pltpu.VMEM((K,), jnp.int32),
        pltpu.VMEM((NW * 16,), jnp.int32),
        pltpu.SemaphoreType.DMA,
        pltpu.SemaphoreType.DMA,
    ],
)
def _ec_pre(u_hbm, v_hbm, rowo_hbm, colo_hbm, cnt_hbm, pre_hbm,
            ru, rv, ridx, cidx, cntv, sem1, sem2):
    w = _wid()
    pltpu.sync_copy(cnt_hbm, cntv)
    basep = _prefix(cntv, w)
    base_w = basep + w * CHB
    padded_w = _my_count(cntv, w)

    def chunk_body(g, _):
        be = pl.multiple_of(base_w + g * K, 8)
        pltpu.sync_copy(rowo_hbm.at[pl.ds(be, K)], ridx)
        pltpu.sync_copy(colo_hbm.at[pl.ds(be, K)], cidx)
        for q in range(K // 16):
            ridx[pl.ds(q * 16, 16)] = jnp.clip(ridx[pl.ds(q * 16, 16)], 0, NP - 1)
            cidx[pl.ds(q * 16, 16)] = jnp.clip(
                cidx[pl.ds(q * 16, 16)] + w * RNG, 0, NP - 1)
        cu = pltpu.async_copy(u_hbm.at[cidx], ru, sem1)
        cv = pltpu.async_copy(v_hbm.at[ridx], rv, sem2)
        cu.wait()
        cv.wait()

        @plsc.parallel_loop(0, K, 1, unroll=2)
        def edge_body(j):
            for f in range(16):
                ix = [_splat(j), _splat(f * 16) + _iota16()]
                plsc.addupdate_scatter(ru, ix, plsc.load_gather(rv, ix))

        ob = pl.multiple_of(basep + g * K, 8)
        pltpu.sync_copy(ru, pre_hbm.at[pl.ds(ob, K)])
        return 0

    lax.fori_loop(0, padded_w // K, chunk_body, 0)


@functools.partial(
    pl.kernel,
    out_type=jax.ShapeDtypeStruct((NP, 256), jnp.float32),
    mesh=_mesh,
    compiler_params=_sc_params,
    scratch_types=[
        pltpu.VMEM((RNG + 1, 256), jnp.float32),
        pltpu.VMEM((K, 256), jnp.float32),
        pltpu.VMEM((K,), jnp.int32),
        pltpu.VMEM((NW * 16,), jnp.int32),
    ],
)
def _ec_max(h_hbm, colo_hbm, cnt_hbm, xe_hbm, acc, rows, cidx, cntv):
    w = _wid()
    pltpu.sync_copy(cnt_hbm, cntv)
    basep = _prefix(cntv, w)
    base_w = basep + w * CHB
    padded_w = _my_count(cntv, w)

    z = jnp.zeros((16,), jnp.float32)
    for r in range(RNG + 1):
        for f in range(16):
            acc[r, pl.ds(f * 16, 16)] = z

    def chunk_body(g, _):
        be = pl.multiple_of(base_w + g * K, 8)
        pltpu.sync_copy(colo_hbm.at[pl.ds(be, K)], cidx)
        hb = pl.multiple_of(basep + g * K, 8)
        pltpu.sync_copy(h_hbm.at[pl.ds(hb, K)], rows)

        def edge_body(j, _):
            csp = plsc.load_gather(cidx, [_splat(j)])
            for f in range(16):
                fx = _splat(f * 16) + _iota16()
                a = plsc.load_gather(acc, [csp, fx])
                v = plsc.load_gather(rows, [_splat(j), fx])
                plsc.store_scatter(acc, [csp, fx], jnp.maximum(a, v))
            return 0

        lax.fori_loop(0, K, edge_body, 0)
        return 0

    lax.fori_loop(0, padded_w // K, chunk_body, 0)
    pltpu.sync_copy(acc.at[pl.ds(0, RNG)], xe_hbm.at[pl.ds(w * RNG, RNG)])


def _dotf(a, b):
    return lax.dot(a, b, precision=lax.Precision.HIGHEST,
                   preferred_element_type=jnp.float32)


def _mm_body(pre_mode, post_scale, out_bias, x_ref, w_ref, bp_ref, bo_ref,
             dv_ref, o_ref):
    x = x_ref[...]
    if pre_mode == 1:
        x = jnp.maximum(x * dv_ref[..., 0:1] + bp_ref[...], 0.0)
    elif pre_mode == 2:
        x = jnp.maximum(x, 0.0)
    y = _dotf(x, w_ref[...])
    if post_scale:
        y = y * dv_ref[..., 0:1]
    if out_bias:
        y = y + bo_ref[...]
    o_ref[...] = y


def _mm(x, w, b_prev=None, b_out=None, dinv=None, pre_mode=0,
        post_scale=False):
    m, kd = x.shape
    f = w.shape[1]
    assert m % 512 == 0
    bp = jnp.zeros((1, f), jnp.float32) if b_prev is None else b_prev.reshape(1, f)
    bo = jnp.zeros((1, f), jnp.float32) if b_out is None else b_out.reshape(1, f)
    dv = (jnp.zeros((512, 128), jnp.float32) if dinv is None else dinv)
    dspec = (pl.BlockSpec((512, 128), lambda i: (0, 0)) if dinv is None
             else pl.BlockSpec((512, 128), lambda i: (i, 0)))
    body = functools.partial(_mm_body, pre_mode, post_scale, b_out is not None)
    return pl.pallas_call(
        body,
        grid=(m // 512,),
        in_specs=[
            pl.BlockSpec((512, kd), lambda i: (i, 0)),
            pl.BlockSpec((kd, f), lambda i: (0, 0)),
            pl.BlockSpec((1, f), lambda i: (0, 0)),
            pl.BlockSpec((1, f), lambda i: (0, 0)),
            dspec,
        ],
        out_specs=pl.BlockSpec((512, f), lambda i: (i, 0)),
        out_shape=jax.ShapeDtypeStruct((m, f), jnp.float32),
    )(x, w, bp, bo, dv)


def _rsqrt_body(d_ref, o_ref):
    o_ref[...] = lax.rsqrt(d_ref[...])


def _rsqrt(deg128):
    return pl.pallas_call(
        _rsqrt_body,
        grid=(NP // 512,),
        in_specs=[pl.BlockSpec((512, 128), lambda i: (i, 0))],
        out_specs=pl.BlockSpec((512, 128), lambda i: (i, 0)),
        out_shape=jax.ShapeDtypeStruct((NP, 128), jnp.float32),
    )(deg128)


def _pool_body(s4_ref, xe_ref, dv_ref, b4_ref, bt_ref, fw_ref, fb_ref,
               ow_ref, ob_ref, o_ref, acc):
    i = pl.program_id(0)

    @pl.when(i == 0)
    def _():
        acc[...] = jnp.zeros_like(acc)

    xg = jnp.maximum(s4_ref[...] * dv_ref[..., 0:1] + b4_ref[...], 0.0)
    xe = xe_ref[...]
    bt = bt_ref[0, 0, :]
    onehot = (bt[None, :] == lax.broadcasted_iota(jnp.int32, (G, 512), 0)
              ).astype(jnp.float32)
    acc[:, 0:256] += _dotf(onehot, xg)
    acc[:, 256:512] += _dotf(onehot, xe)

    @pl.when(i == NP // 512 - 1)
    def _():
        g = jnp.maximum(_dotf(acc[...], fw_ref[...]) + fb_ref[...], 0.0)
        o_ref[...] = _dotf(g, ow_ref[...]) + ob_ref[0, 0]


def _pool(s4, xe, dinv, b4, batch3, fc1_w, fc1_b, out_w, out_b):
    return pl.pallas_call(
        _pool_body,
        grid=(NP // 512,),
        in_specs=[
            pl.BlockSpec((512, 256), lambda i: (i, 0)),
            pl.BlockSpec((512, 256), lambda i: (i, 0)),
            pl.BlockSpec((512, 128), lambda i: (i, 0)),
            pl.BlockSpec((1, 256), lambda i: (0, 0)),
            pl.BlockSpec((1, 1, 512), lambda i: (i, 0, 0)),
            pl.BlockSpec((512, 256), lambda i: (0, 0)),
            pl.BlockSpec((1, 256), lambda i: (0, 0)),
            pl.BlockSpec((256, 128), lambda i: (0, 0)),
            pl.BlockSpec((1, 1), lambda i: (0, 0)),
        ],
        out_specs=pl.BlockSpec((G, 128), lambda i: (0, 0)),
        out_shape=jax.ShapeDtypeStruct((G, 128), jnp.float32),
        scratch_shapes=[pltpu.VMEM((G, 512), jnp.float32)],
    )(s4, xe, dinv, b4.reshape(1, 256), batch3, fc1_w, fc1_b.reshape(1, 256),
      jnp.broadcast_to(out_w, (256, 128)), out_b.reshape(1, 1))


def kernel(x, edge_index, batch, gcn1_w, gcn1_b, gcn2_w, gcn2_b, gcn3_w, gcn3_b, gcn4_w, gcn4_b, ecn1_w1, ecn1_b1, ecn1_w2, ecn1_b2, ecn2_w1, ecn2_b1, ecn2_w2, ecn2_b2, fc1_w, fc1_b, out_w, out_b):
    row, col = edge_index[0], edge_index[1]

    cnts = _filter_count(col)
    rowo, colo = _filter_compact(row, col, cnts)
    deg = _deg_kernel(colo, cnts)
    dinv = _rsqrt(jnp.broadcast_to(deg[:, None], (NP, 128)))

    xp = jnp.zeros((NP, 128), jnp.float32).at[:N].set(x)

    y = _mm(xp, gcn1_w, dinv=dinv, post_scale=True)
    s = _gcn_agg(y, rowo, colo, cnts)
    for (w_l, b_prev) in ((gcn2_w, gcn1_b), (gcn3_w, gcn2_b), (gcn4_w, gcn3_b)):
        y = _mm(s, w_l, b_prev=b_prev, dinv=dinv, pre_mode=1, post_scale=True)
        s = _gcn_agg(y, rowo, colo, cnts)

    def edge_conv(src, w1, b1, w2, b2):
        fin = src.shape[1]
        u = _mm(src, w1[:fin] - w1[fin:], b_out=b1)
        v = _mm(src, w1[fin:])
        pre = _ec_pre(u, v, rowo, colo, cnts)
        h = _mm(pre, w2, b_out=b2, pre_mode=2)
        return _ec_max(h, colo, cnts)

    xe = edge_conv(xp, ecn1_w1, ecn1_b1, ecn1_w2, ecn1_b2)
    xe = edge_conv(xe, ecn2_w1, ecn2_b1, ecn2_w2, ecn2_b2)

    batch_p = jnp.full((NP,), G + 7, jnp.int32).at[:N].set(batch.astype(jnp.int32))
    out = _pool(s, xe, dinv, gcn4_b, batch_p.reshape(NP // 512, 1, 512),
                fc1_w, fc1_b, out_w, out_b)
    return out[:, :1]


# speedup vs baseline: 2.4103x; 1.0112x over previous
import functools

import jax
import jax.numpy as jnp
from jax import lax
from jax.experimental import pallas as pl
from jax.experimental.pallas import tpu as pltpu
from jax.experimental.pallas import tpu_sc as plsc

N = 10000
E = 320000
G = 64

NW = 32
RNG = 320
NP = NW * RNG
CH = 4000
CHB = CH + 16
K = 128
NCHK = E // CH
EBUFP = E + NW * (7 * NCHK + K)
EBUF = EBUFP + NW * CHB

_mesh = plsc.VectorSubcoreMesh(core_axis_name="c", subcore_axis_name="s")
_sc_params = pltpu.CompilerParams(needs_layout_passes=False)


def _wid():
    return lax.axis_index("s") * 2 + lax.axis_index("c")


def _iota16():
    return lax.iota(jnp.int32, 16)


def _splat(x):
    return jnp.full((16,), x, jnp.int32)


def _scal(v):
    return jnp.max(v)


def _prefix(cntv, w):
    base_v = jnp.zeros((16,), jnp.int32)
    for t in range(NW):
        ct = cntv[pl.ds(t * 16, 16)]
        base_v = base_v + jnp.where(t < w, ct, 0)
    return _scal(base_v)


def _my_count(cntv, w):
    return _scal(plsc.load_gather(cntv, [_splat(w * 16) + _iota16()]))


@functools.partial(
    pl.kernel,
    out_type=jax.ShapeDtypeStruct((NW * 16,), jnp.int32),
    mesh=_mesh,
    compiler_params=_sc_params,
    scratch_types=[pltpu.VMEM((CH,), jnp.int32),
                   pltpu.VMEM((16,), jnp.int32)],
)
def _filter_count(col_hbm, cnt_hbm, colv, outv):
    w = _wid()
    lo = w * RNG
    hi = lo + RNG

    def chunk_body(ci, padded):
        pltpu.sync_copy(col_hbm.at[pl.ds(ci * CH, CH)], colv)

        def grp_body(gi, c):
            cv = plsc.load_gather(colv, [_splat(gi * 16) + _iota16()])
            m = (cv >= lo) & (cv < hi)
            return c + _scal(plsc.all_reduce_population_count(m))

        c = lax.fori_loop(0, CH // 16, grp_body, jnp.int32(0))
        return padded + ((c + 7) // 8) * 8

    padded = lax.fori_loop(0, NCHK, chunk_body, jnp.int32(0))
    padded = ((padded + K - 1) // K) * K
    outv[...] = jnp.broadcast_to(padded, (16,))
    pltpu.sync_copy(outv, cnt_hbm.at[pl.ds(w * 16, 16)])


@functools.partial(
    pl.kernel,
    out_type=(
        jax.ShapeDtypeStruct((EBUF,), jnp.int32),
        jax.ShapeDtypeStruct((EBUF,), jnp.int32),
    ),
    mesh=_mesh,
    compiler_params=_sc_params,
    scratch_types=[
        pltpu.VMEM((CH,), jnp.int32),
        pltpu.VMEM((CH,), jnp.int32),
        pltpu.VMEM((NW * 16,), jnp.int32),
        pltpu.VMEM((CHB,), jnp.int32),
        pltpu.VMEM((CHB,), jnp.int32),
    ],
)
def _filter_compact(row_hbm, col_hbm, cnt_hbm, rowo_hbm, colo_hbm,
                    rowv, colv, cntv, rbuf, cbuf):
    w = _wid()
    lo = w * RNG
    hi = lo + RNG
    pltpu.sync_copy(cnt_hbm, cntv)
    glob0 = _prefix(cntv, w) + w * CHB

    def chunk_body(ci, glob):
        pltpu.sync_copy(col_hbm.at[pl.ds(ci * CH, CH)], colv)
        pltpu.sync_copy(row_hbm.at[pl.ds(ci * CH, CH)], rowv)

        def grp_body(gi, loc):
            idx = _splat(gi * 16) + _iota16()
            cv = plsc.load_gather(colv, [idx])
            rv = plsc.load_gather(rowv, [idx])
            m = (cv >= lo) & (cv < hi)
            plsc.store_compressed(rbuf.at[pl.ds(loc, 16)], rv, mask=m)
            plsc.store_compressed(cbuf.at[pl.ds(loc, 16)], cv - lo, mask=m)
            return loc + _scal(plsc.all_reduce_population_count(m))

        loc = lax.fori_loop(0, CH // 16, grp_body, jnp.int32(0))
        p = (8 - lax.rem(loc, 8)) & 7
        pm = _iota16() < p
        plsc.store_scatter(rbuf, [_splat(loc) + _iota16()],
                           jnp.zeros((16,), jnp.int32), mask=pm)
        plsc.store_scatter(cbuf, [_splat(loc) + _iota16()],
                           _splat(RNG), mask=pm)
        globa = pl.multiple_of(glob, 8)
        pltpu.sync_copy(rbuf, rowo_hbm.at[pl.ds(globa, CHB)])
        pltpu.sync_copy(cbuf, colo_hbm.at[pl.ds(globa, CHB)])
        return glob + loc + p

    glob = lax.fori_loop(0, NCHK, chunk_body, glob0)
    for q in range(K // 16):
        rbuf[pl.ds(q * 16, 16)] = jnp.zeros((16,), jnp.int32)
        cbuf[pl.ds(q * 16, 16)] = _splat(RNG)
    globa = pl.multiple_of(glob, 8)
    pltpu.sync_copy(rbuf.at[pl.ds(0, K)], rowo_hbm.at[pl.ds(globa, K)])
    pltpu.sync_copy(cbuf.at[pl.ds(0, K)], colo_hbm.at[pl.ds(globa, K)])


@functools.partial(
    pl.kernel,
    out_type=jax.ShapeDtypeStruct((NP, 256), jnp.float32),
    mesh=_mesh,
    compiler_params=_sc_params,
    scratch_types=[
        pltpu.VMEM((RNG + 1, 256), jnp.float32),
        pltpu.VMEM((K, 256), jnp.float32),
        pltpu.VMEM((K,), jnp.int32),
        pltpu.VMEM((K,), jnp.int32),
        pltpu.VMEM((NW * 16,), jnp.int32),
        pltpu.SemaphoreType.DMA,
    ],
)
def _gcn_agg(y_hbm, rowo_hbm, colo_hbm, cnt_hbm, out_hbm,
             acc, rows, ridx, cidx, cntv, sem):
    w = _wid()
    pltpu.sync_copy(cnt_hbm, cntv)
    base_w = _prefix(cntv, w) + w * CHB
    padded_w = _my_count(cntv, w)

    pltpu.sync_copy(y_hbm.at[pl.ds(w * RNG, RNG)], acc.at[pl.ds(0, RNG)])

    def chunk_body(g, _):
        be = pl.multiple_of(base_w + g * K, 8)
        pltpu.sync_copy(rowo_hbm.at[pl.ds(be, K)], ridx)
        pltpu.sync_copy(colo_hbm.at[pl.ds(be, K)], cidx)
        for q in range(K // 16):
            ridx[pl.ds(q * 16, 16)] = jnp.clip(ridx[pl.ds(q * 16, 16)], 0, NP - 1)
        pltpu.async_copy(y_hbm.at[ridx], rows, sem).wait()

        @plsc.parallel_loop(0, K, 1, unroll=4)
        def edge_body(j):
            csp = plsc.load_gather(cidx, [_splat(j)])
            for f in range(16):
                v = plsc.load_gather(rows, [_splat(j), _splat(f * 16) + _iota16()])
                plsc.addupdate_scatter(acc, [csp, _splat(f * 16) + _iota16()], v)

        return 0

    lax.fori_loop(0, padded_w // K, chunk_body, 0)
    pltpu.sync_copy(acc.at[pl.ds(0, RNG)], out_hbm.at[pl.ds(w * RNG, RNG)])


@functools.partial(
    pl.kernel,
    out_type=jax.ShapeDtypeStruct((NP,), jnp.float32),
    mesh=_mesh,
    compiler_params=_sc_params,
    scratch_types=[
        pltpu.VMEM((RNG + 16,), jnp.float32),
        pltpu.VMEM((K,), jnp.int32),
        pltpu.VMEM((NW * 16,), jnp.int32),
    ],
)
def _deg_kernel(colo_hbm, cnt_hbm, deg_hbm, acc, cidx, cntv):
    w = _wid()
    pltpu.sync_copy(cnt_hbm, cntv)
    base_w = _prefix(cntv, w) + w * CHB
    padded_w = _my_count(cntv, w)

    ones = jnp.ones((16,), jnp.float32)
    for q in range((RNG + 16) // 16):
        acc[pl.ds(q * 16, 16)] = ones
    m0 = _iota16() < 1

    def chunk_body(g, _):
        be = pl.multiple_of(base_w + g * K, 8)
        pltpu.sync_copy(colo_hbm.at[pl.ds(be, K)], cidx)

        @plsc.parallel_loop(0, K, 1, unroll=4)
        def edge_body(j):
            csp = plsc.load_gather(cidx, [_splat(j)])
            plsc.addupdate_scatter(acc, [csp], ones, mask=m0)

        return 0

    lax.fori_loop(0, padded_w // K, chunk_body, 0)
    pltpu.sync_copy(acc.at[pl.ds(0, RNG)], deg_hbm.at[pl.ds(w * RNG, RNG)])


@functools.partial(
    pl.kernel,
    out_type=jax.ShapeDtypeStruct((EBUFP, 256), jnp.float32),
    mesh=_mesh,
    compiler_params=_sc_params,
    scratch_types=[
        pltpu.VMEM((K, 256), jnp.float32),
        pltpu.VMEM((K, 256), jnp.float32),
        pltpu.VMEM((K,), jnp.int32),
        pltpu.VMEM((K,), jnp.int32),
        pltpu.VMEM((NW * 16,), jnp.int32),
        pltpu.SemaphoreType.DMA,
        pltpu.SemaphoreType.DMA,
    ],
)
def _ec_pre(u_hbm, v_hbm, rowo_hbm, colo_hbm, cnt_hbm, pre_hbm,
            ru, rv, ridx, cidx, cntv, sem1, sem2):
    w = _wid()
    pltpu.sync_copy(cnt_hbm, cntv)
    basep = _prefix(cntv, w)
    base_w = basep + w * CHB
    padded_w = _my_count(cntv, w)

    def chunk_body(g, _):
        be = pl.multiple_of(base_w + g * K, 8)
        pltpu.sync_copy(rowo_hbm.at[pl.ds(be, K)], ridx)
        pltpu.sync_copy(colo_hbm.at[pl.ds(be, K)], cidx)
        for q in range(K // 16):
            ridx[pl.ds(q * 16, 16)] = jnp.clip(ridx[pl.ds(q * 16, 16)], 0, NP - 1)
            cidx[pl.ds(q * 16, 16)] = jnp.clip(
                cidx[pl.ds(q * 16, 16)] + w * RNG, 0, NP - 1)
        cu = pltpu.async_copy(u_hbm.at[cidx], ru, sem1)
        cv = pltpu.async_copy(v_hbm.at[ridx], rv, sem2)
        cu.wait()
        cv.wait()

        @plsc.parallel_loop(0, K, 1, unroll=2)
        def edge_body(j):
            for f in range(16):
                ix = [_splat(j), _splat(f * 16) + _iota16()]
                plsc.addupdate_scatter(ru, ix, plsc.load_gather(rv, ix))

        ob = pl.multiple_of(basep + g * K, 8)
        pltpu.sync_copy(ru, pre_hbm.at[pl.ds(ob, K)])
        return 0

    lax.fori_loop(0, padded_w // K, chunk_body, 0)


@functools.partial(
    pl.kernel,
    out_type=jax.ShapeDtypeStruct((NP, 256), jnp.float32),
    mesh=_mesh,
    compiler_params=_sc_params,
    scratch_types=[
        pltpu.VMEM((RNG + 1, 256), jnp.float32),
        pltpu.VMEM((K, 256), jnp.float32),
        pltpu.VMEM((K,), jnp.int32),
        pltpu.VMEM((NW * 16,), jnp.int32),
    ],
)
def _ec_max(h_hbm, colo_hbm, cnt_hbm, xe_hbm, acc, rows, cidx, cntv):
    w = _wid()
    pltpu.sync_copy(cnt_hbm, cntv)
    basep = _prefix(cntv, w)
    base_w = basep + w * CHB
    padded_w = _my_count(cntv, w)

    z = jnp.zeros((16,), jnp.float32)
    for r in range(RNG + 1):
        for f in range(16):
            acc[r, pl.ds(f * 16, 16)] = z

    def chunk_body(g, _):
        be = pl.multiple_of(base_w + g * K, 8)
        pltpu.sync_copy(colo_hbm.at[pl.ds(be, K)], cidx)
        hb = pl.multiple_of(basep + g * K, 8)
        pltpu.sync_copy(h_hbm.at[pl.ds(hb, K)], rows)

        def edge_body(j, _):
            csp = plsc.load_gather(cidx, [_splat(j)])
            for f in range(16):
                fx = _splat(f * 16) + _iota16()
                a = plsc.load_gather(acc, [csp, fx])
                v = plsc.load_gather(rows, [_splat(j), fx])
                plsc.store_scatter(acc, [csp, fx], jnp.maximum(a, v))
            return 0

        lax.fori_loop(0, K, edge_body, 0)
        return 0

    lax.fori_loop(0, padded_w // K, chunk_body, 0)
    pltpu.sync_copy(acc.at[pl.ds(0, RNG)], xe_hbm.at[pl.ds(w * RNG, RNG)])


def _dotf(a, b):
    return lax.dot(a, b, precision=lax.Precision.HIGHEST,
                   preferred_element_type=jnp.float32)


def _mm_body(pre_mode, post_scale, out_bias, x_ref, w_ref, bp_ref, bo_ref,
             dv_ref, o_ref):
    x = x_ref[...]
    if pre_mode == 1:
        x = jnp.maximum(x * dv_ref[..., 0:1] + bp_ref[...], 0.0)
    elif pre_mode == 2:
        x = jnp.maximum(x, 0.0)
    y = _dotf(x, w_ref[...])
    if post_scale:
        y = y * dv_ref[..., 0:1]
    if out_bias:
        y = y + bo_ref[...]
    o_ref[...] = y


def _mm(x, w, b_prev=None, b_out=None, dinv=None, pre_mode=0,
        post_scale=False):
    m, kd = x.shape
    f = w.shape[1]
    assert m % 512 == 0
    bp = jnp.zeros((1, f), jnp.float32) if b_prev is None else b_prev.reshape(1, f)
    bo = jnp.zeros((1, f), jnp.float32) if b_out is None else b_out.reshape(1, f)
    dv = (jnp.zeros((512, 128), jnp.float32) if dinv is None else dinv)
    dspec = (pl.BlockSpec((512, 128), lambda i: (0, 0)) if dinv is None
             else pl.BlockSpec((512, 128), lambda i: (i, 0)))
    body = functools.partial(_mm_body, pre_mode, post_scale, b_out is not None)
    return pl.pallas_call(
        body,
        grid=(m // 512,),
        in_specs=[
            pl.BlockSpec((512, kd), lambda i: (i, 0)),
            pl.BlockSpec((kd, f), lambda i: (0, 0)),
            pl.BlockSpec((1, f), lambda i: (0, 0)),
            pl.BlockSpec((1, f), lambda i: (0, 0)),
            dspec,
        ],
        out_specs=pl.BlockSpec((512, f), lambda i: (i, 0)),
        out_shape=jax.ShapeDtypeStruct((m, f), jnp.float32),
    )(x, w, bp, bo, dv)


def _rsqrt_body(d_ref, o_ref):
    o_ref[...] = lax.rsqrt(d_ref[...])


def _rsqrt(deg128):
    return pl.pallas_call(
        _rsqrt_body,
        grid=(NP // 512,),
        in_specs=[pl.BlockSpec((512, 128), lambda i: (i, 0))],
        out_specs=pl.BlockSpec((512, 128), lambda i: (i, 0)),
        out_shape=jax.ShapeDtypeStruct((NP, 128), jnp.float32),
    )(deg128)


def _pool_body(s4_ref, xe_ref, dv_ref, b4_ref, bt_ref, fw_ref, fb_ref,
               ow_ref, ob_ref, o_ref, acc):
    i = pl.program_id(0)

    @pl.when(i == 0)
    def _():
        acc[...] = jnp.zeros_like(acc)

    xg = jnp.maximum(s4_ref[...] * dv_ref[..., 0:1] + b4_ref[...], 0.0)
    xe = xe_ref[...]
    bt = bt_ref[0, 0, :]
    onehot = (bt[None, :] == lax.broadcasted_iota(jnp.int32, (G, 512), 0)
              ).astype(jnp.float32)
    acc[:, 0:256] += _dotf(onehot, xg)
    acc[:, 256:512] += _dotf(onehot, xe)

    @pl.when(i == NP // 512 - 1)
    def _():
        g = jnp.maximum(_dotf(acc[...], fw_ref[...]) + fb_ref[...], 0.0)
        o_ref[...] = _dotf(g, ow_ref[...]) + ob_ref[0, 0]


def _pool(s4, xe, dinv, b4, batch3, fc1_w, fc1_b, out_w, out_b):
    return pl.pallas_call(
        _pool_body,
        grid=(NP // 512,),
        in_specs=[
            pl.BlockSpec((512, 256), lambda i: (i, 0)),
            pl.BlockSpec((512, 256), lambda i: (i, 0)),
            pl.BlockSpec((512, 128), lambda i: (i, 0)),
            pl.BlockSpec((1, 256), lambda i: (0, 0)),
            pl.BlockSpec((1, 1, 512), lambda i: (i, 0, 0)),
            pl.BlockSpec((512, 256), lambda i: (0, 0)),
            pl.BlockSpec((1, 256), lambda i: (0, 0)),
            pl.BlockSpec((256, 128), lambda i: (0, 0)),
            pl.BlockSpec((1, 1), lambda i: (0, 0)),
        ],
        out_specs=pl.BlockSpec((G, 128), lambda i: (0, 0)),
        out_shape=jax.ShapeDtypeStruct((G, 128), jnp.float32),
        scratch_shapes=[pltpu.VMEM((G, 512), jnp.float32)],
    )(s4, xe, dinv, b4.reshape(1, 256), batch3, fc1_w, fc1_b.reshape(1, 256),
      jnp.broadcast_to(out_w, (256, 128)), out_b.reshape(1, 1))


def kernel(x, edge_index, batch, gcn1_w, gcn1_b, gcn2_w, gcn2_b, gcn3_w, gcn3_b, gcn4_w, gcn4_b, ecn1_w1, ecn1_b1, ecn1_w2, ecn1_b2, ecn2_w1, ecn2_b1, ecn2_w2, ecn2_b2, fc1_w, fc1_b, out_w, out_b):
    row, col = edge_index[0], edge_index[1]

    cnts = _filter_count(col)
    rowo, colo = _filter_compact(row, col, cnts)
    deg = _deg_kernel(colo, cnts)
    dinv = _rsqrt(jnp.broadcast_to(deg[:, None], (NP, 128)))

    xp = jnp.zeros((NP, 128), jnp.float32).at[:N].set(x)

    y = _mm(xp, gcn1_w, dinv=dinv, post_scale=True)
    s = _gcn_agg(y, rowo, colo, cnts)
    for (w_l, b_prev) in ((gcn2_w, gcn1_b), (gcn3_w, gcn2_b), (gcn4_w, gcn3_b)):
        y = _mm(s, w_l, b_prev=b_prev, dinv=dinv, pre_mode=1, post_scale=True)
        s = _gcn_agg(y, rowo, colo, cnts)

    def edge_conv(src, w1, b1, w2, b2):
        fin = src.shape[1]
        u = _mm(src, w1[:fin] - w1[fin:], b_out=b1)
        v = _mm(src, w1[fin:])
        pre = _ec_pre(u, v, rowo, colo, cnts)
        h = _mm(pre, w2, b_out=b2, pre_mode=2)
        return _ec_max(h, colo, cnts)

    xe = edge_conv(xp, ecn1_w1, ecn1_b1, ecn1_w2, ecn1_b2)
    xe = edge_conv(xe, ecn2_w1, ecn2_b1, ecn2_w2, ecn2_b2)

    batch_p = jnp.full((NP,), G + 7, jnp.int32).at[:N].set(batch.astype(jnp.int32))
    out = _pool(s, xe, dinv, gcn4_b, batch_p.reshape(NP // 512, 1, 512),
                fc1_w, fc1_b, out_w, out_b)
    return out[:, :1]
